# baseline (speedup 1.0000x reference)
import functools

import jax
import jax.numpy as jnp
from jax import lax
from jax.experimental import pallas as pl
from jax.experimental.pallas import tpu as pltpu
from jax.experimental.pallas import tpu_sc as plsc

MAX_FLARES = 6
NMS_KERNEL = 9
CONF_THRESHOLD = 0.3


def _poshead_one(hm, sc_ref):
    H, W = hm.shape
    pad = NMS_KERNEL // 2
    neg = jnp.float32(-jnp.inf)

    xp = jnp.concatenate(
        [jnp.full((pad, W), neg, jnp.float32), hm, jnp.full((pad, W), neg, jnp.float32)],
        axis=0,
    )
    s1 = jnp.maximum(jax.lax.slice(xp, (0, 0), (H + 7, W)),
                     jax.lax.slice(xp, (1, 0), (H + 8, W)))
    s2 = jnp.maximum(jax.lax.slice(s1, (0, 0), (H + 5, W)),
                     jax.lax.slice(s1, (2, 0), (H + 7, W)))
    s3 = jnp.maximum(jax.lax.slice(s2, (0, 0), (H + 1, W)),
                     jax.lax.slice(s2, (4, 0), (H + 5, W)))
    rowp = jnp.maximum(jax.lax.slice(s3, (0, 0), (H, W)),
                       jax.lax.slice(xp, (8, 0), (H + 8, W)))

    yp = jnp.concatenate(
        [jnp.full((H, pad), neg, jnp.float32), rowp, jnp.full((H, pad), neg, jnp.float32)],
        axis=1,
    )
    t1 = jnp.maximum(jax.lax.slice(yp, (0, 0), (H, W + 7)),
                     jax.lax.slice(yp, (0, 1), (H, W + 8)))
    t2 = jnp.maximum(jax.lax.slice(t1, (0, 0), (H, W + 5)),
                     jax.lax.slice(t1, (0, 2), (H, W + 7)))
    t3 = jnp.maximum(jax.lax.slice(t2, (0, 0), (H, W + 1)),
                     jax.lax.slice(t2, (0, 4), (H, W + 5)))
    pooled = jnp.maximum(jax.lax.slice(t3, (0, 0), (H, W)),
                         jax.lax.slice(yp, (0, 8), (H, W + 8)))

    peaks = (hm == pooled) & (hm > CONF_THRESHOLD)
    scores = jnp.where(peaks, hm, 0.0)
    sc_ref[...] = scores
    rowmax = jnp.max(scores, axis=1, keepdims=True)

    riota = jax.lax.broadcasted_iota(jnp.int32, (H, 1), 0)
    ciota = jax.lax.broadcasted_iota(jnp.int32, (1, W), 1)
    big = jnp.int32(2**30)
    inv_step = jnp.float32(1.0) / jnp.float32(W - 1)

    rix = jax.lax.broadcasted_iota(jnp.int32, (8, 128), 0)
    cix = jax.lax.broadcasted_iota(jnp.int32, (8, 128), 1)
    out = jnp.zeros((8, 128), jnp.float32)
    for k in range(MAX_FLARES):
        m = jnp.max(rowmax)
        r = jnp.min(jnp.where(rowmax == m, riota, big))
        row = sc_ref[pl.ds(r, 1), :]
        c = jnp.min(jnp.where(row == m, ciota, big))
        valid = m >= CONF_THRESHOLD
        x = jnp.where(valid, c.astype(jnp.float32) * inv_step, -1.0)
        y = jnp.where(valid, r.astype(jnp.float32) * inv_step, -1.0)
        conf = jnp.where(valid, m, 0.0)
        sel = cix == k
        out = jnp.where((rix == 0) & sel, conf, out)
        out = jnp.where((rix == 1) & sel, x, out)
        out = jnp.where((rix == 2) & sel, y, out)
        if k + 1 < MAX_FLARES:
            newrow = jnp.where(ciota == c, 0.0, row)
            sc_ref[pl.ds(r, 1), :] = newrow
            rowmax = jnp.where(riota == r, jnp.max(newrow), rowmax)
    return out


N_PER = 4


L = 16
SC_IMGS = 4
RPT = 64
SLAB = 32
NV = 512 // L
SC_W = 512
_BIGI = 1 << 30


HPW = SC_W + 2 * L


def _sc_body(hm, out, inbuf, hpad, scb, cnd, csh, mrg, obuf):
    c = lax.axis_index("c")
    s = lax.axis_index("s")
    img = 2 * c + s // 8
    srow = (s % 8) * RPT

    lane = lax.iota(jnp.int32, L)
    neg = jnp.full((L,), -jnp.inf, jnp.float32)
    zero = jnp.zeros((L,), jnp.float32)

    for j_ in range(SLAB):
        hpad[pl.ds(j_ * HPW, L)] = neg
        hpad[pl.ds(j_ * HPW + L + SC_W, L)] = neg

    rmx = [zero, zero, zero, zero]
    for slab in range(2):
        base = srow + slab * SLAB
        if slab == 0:
            edge = s % 8 == 0

            @pl.when(edge)
            def _():
                pltpu.sync_copy(hm.at[pl.ds(img * SC_W * SC_W, 40 * SC_W)],
                                inbuf.at[pl.ds(8 * SC_W, 40 * SC_W)])
                for j_ in range(8):
                    for v_ in range(NV):
                        inbuf[pl.ds(j_ * SC_W + v_ * L, L)] = neg

            @pl.when(jnp.logical_not(edge))
            def _():
                pltpu.sync_copy(hm.at[pl.ds(img * SC_W * SC_W + (base - 8) * SC_W, 48 * SC_W)], inbuf)
        else:
            edge = s % 8 == 7

            @pl.when(edge)
            def _():
                pltpu.sync_copy(hm.at[pl.ds(img * SC_W * SC_W + (base - 8) * SC_W, 40 * SC_W)],
                                inbuf.at[pl.ds(0, 40 * SC_W)])
                for j_ in range(40, 48):
                    for v_ in range(NV):
                        inbuf[pl.ds(j_ * SC_W + v_ * L, L)] = neg

            @pl.when(jnp.logical_not(edge))
            def _():
                pltpu.sync_copy(hm.at[pl.ds(img * SC_W * SC_W + (base - 8) * SC_W, 48 * SC_W)], inbuf)

        def pool_row(j, rmx_c):
            lr = slab * SLAB + j
            for v in range(NV):
                vp = inbuf[pl.ds((j + 4) * SC_W + v * L, L)]
                for dy in range(5, 13):
                    vp = jnp.maximum(vp, inbuf[pl.ds((j + dy) * SC_W + v * L, L)])
                hpad[pl.ds(j * HPW + L + v * L, L)] = vp
            rm = zero
            for v in range(NV):
                off = j * HPW + L + v * L
                p = hpad[pl.ds(off - 4, L)]
                for dd in range(-3, 5):
                    p = jnp.maximum(p, hpad[pl.ds(off + dd, L)])
                h = inbuf[pl.ds((j + 8) * SC_W + v * L, L)]
                sv = jnp.where((h == p) & (h > CONF_THRESHOLD), h, 0.0)
                scb[pl.ds(lr * SC_W + v * L, L)] = sv
                rm = jnp.maximum(rm, sv)
            m = jnp.max(rm)
            ins = lane == lr % L
            g = lr // L
            return tuple(
                jnp.where(ins & (g == gi), m, rmx_c[gi]) for gi in range(4)
            )

        rmx = lax.fori_loop(0, SLAB, pool_row, tuple(rmx))

    cv = zero
    ci = jnp.zeros((L,), jnp.int32)
    g0, g1, g2, g3 = rmx
    for k in range(MAX_FLARES):
        m = jnp.max(jnp.maximum(jnp.maximum(g0, g1), jnp.maximum(g2, g3)))
        rr = jnp.min(jnp.minimum(
            jnp.minimum(jnp.where(g0 == m, lane, 4096),
                        jnp.where(g1 == m, lane + L, 4096)),
            jnp.minimum(jnp.where(g2 == m, lane + 2 * L, 4096),
                        jnp.where(g3 == m, lane + 3 * L, 4096))))
        cmin = jnp.full((L,), 99999, jnp.int32)
        for v in range(NV):
            sv = scb[pl.ds(rr * SC_W + v * L, L)]
            cmin = jnp.minimum(cmin, jnp.where(sv == m, lane + v * L, 99999))
        cc = jnp.min(cmin)
        cv = jnp.where(lane == k, m, cv)
        ci = jnp.where(lane == k, (srow + rr) * SC_W + cc, ci)
        if k + 1 < MAX_FLARES:
            blk = (cc // L) * L
            sv = scb[pl.ds(rr * SC_W + blk, L)]
            scb[pl.ds(rr * SC_W + blk, L)] = jnp.where(lane + blk == cc, 0.0, sv)
            nrm = zero
            for v in range(NV):
                nrm = jnp.maximum(nrm, scb[pl.ds(rr * SC_W + v * L, L)])
            nm = jnp.max(nrm)
            ins = lane == rr % L
            g = rr // L
            g0 = jnp.where(ins & (g == 0), nm, g0)
            g1 = jnp.where(ins & (g == 1), nm, g1)
            g2 = jnp.where(ins & (g == 2), nm, g2)
            g3 = jnp.where(ins & (g == 3), nm, g3)

    cnd[pl.ds(0, L)] = cv
    cnd[pl.ds(L, L)] = plsc.bitcast(ci, jnp.float32)
    pltpu.sync_copy(cnd, csh.at[pl.ds(s * 2 * L, 2 * L)])
    plsc.subcore_barrier()

    @pl.when(s % 8 == 0)
    def _():
        pltpu.sync_copy(csh.at[pl.ds(s * 2 * L, 8 * 2 * L)], mrg)
        mv = [mrg[pl.ds(t * 2 * L, L)] for t in range(8)]
        mi = [plsc.bitcast(mrg[pl.ds(t * 2 * L + L, L)], jnp.int32) for t in range(8)]
        confv = zero
        xv = zero
        yv = zero
        inv_step = jnp.float32(1.0) / jnp.float32(SC_W - 1)
        for k in range(MAX_FLARES):
            acc = mv[0]
            for t in range(1, 8):
                acc = jnp.maximum(acc, mv[t])
            m = jnp.max(acc)
            iacc = jnp.where(mv[0] == m, mi[0], _BIGI)
            for t in range(1, 8):
                iacc = jnp.minimum(iacc, jnp.where(mv[t] == m, mi[t], _BIGI))
            li = jnp.min(iacc)
            valid = m >= CONF_THRESHOLD
            confv = jnp.where(lane == k, jnp.where(valid, m, 0.0), confv)
            xs = (li % SC_W).astype(jnp.float32) * inv_step
            ys = (li // SC_W).astype(jnp.float32) * inv_step
            xv = jnp.where(lane == k, jnp.where(valid, xs, -1.0), xv)
            yv = jnp.where(lane == k, jnp.where(valid, ys, -1.0), yv)
            if k + 1 < MAX_FLARES:
                mv = [jnp.where(mi[t] == li, -1.0, mv[t]) for t in range(8)]
        obuf[0, pl.ds(0, L)] = confv
        obuf[1, pl.ds(0, L)] = xv
        obuf[2, pl.ds(0, L)] = yv
        pltpu.sync_copy(obuf, out.at[img])


@functools.partial(
    pl.kernel,
    mesh=plsc.VectorSubcoreMesh(core_axis_name="c", subcore_axis_name="s"),
    compiler_params=pltpu.CompilerParams(needs_layout_passes=False),
    out_type=jax.ShapeDtypeStruct((SC_IMGS, 8, 128), jnp.float32),
    scratch_types=[
        pltpu.VMEM((48 * SC_W,), jnp.float32),
        pltpu.VMEM((SLAB * HPW,), jnp.float32),
        pltpu.VMEM((RPT * SC_W,), jnp.float32),
        pltpu.VMEM((2 * L,), jnp.float32),
        pltpu.VMEM_SHARED((L * 2 * L,), jnp.float32),
        pltpu.VMEM((8 * 2 * L,), jnp.float32),
        pltpu.VMEM((8, 128), jnp.float32),
    ],
)
def _sc_call(hm, out, inbuf, hpad, scb, cnd, csh, mrg, obuf):
    _sc_body(hm, out, inbuf, hpad, scb, cnd, csh, mrg, obuf)


def _poshead_kernel(hm_ref, out_ref, sc_ref):
    for j in range(N_PER):
        out_ref[j] = _poshead_one(hm_ref[j], sc_ref.at[j])


@jax.jit
def kernel(heatmap):
    B, _, H, W = heatmap.shape
    hm = heatmap.reshape(B, H, W)
    sc_out = _sc_call(hm.reshape(-1))
    n_tc = B - SC_IMGS
    tc_out = pl.pallas_call(
        _poshead_kernel,
        grid=(n_tc // N_PER,),
        in_specs=[pl.BlockSpec((N_PER, H, W), lambda b: (b + SC_IMGS // N_PER, 0, 0))],
        out_specs=pl.BlockSpec((N_PER, 8, 128), lambda b: (b, 0, 0)),
        out_shape=jax.ShapeDtypeStruct((n_tc, 8, 128), jnp.float32),
        scratch_shapes=[pltpu.VMEM((N_PER, H, W), jnp.float32)],
    )(hm)
    out = jnp.concatenate([sc_out, tc_out], axis=0)
    conf = out[:, 0, :MAX_FLARES]
    pos = jnp.stack([out[:, 1, :MAX_FLARES], out[:, 2, :MAX_FLARES]], axis=-1)
    return pos, conf


# speedup vs baseline: 1.2211x; 1.2211x over previous
import functools

import jax
import jax.numpy as jnp
from jax import lax
from jax.experimental import pallas as pl
from jax.experimental.pallas import tpu as pltpu
from jax.experimental.pallas import tpu_sc as plsc

MAX_FLARES = 6
NMS_KERNEL = 9
CONF_THRESHOLD = 0.3


def _poshead_one(hm, sc_ref):
    H, W = hm.shape
    pad = NMS_KERNEL // 2
    neg = jnp.float32(-jnp.inf)

    xp = jnp.concatenate(
        [jnp.full((pad, W), neg, jnp.float32), hm, jnp.full((pad, W), neg, jnp.float32)],
        axis=0,
    )
    s1 = jnp.maximum(jax.lax.slice(xp, (0, 0), (H + 7, W)),
                     jax.lax.slice(xp, (1, 0), (H + 8, W)))
    s2 = jnp.maximum(jax.lax.slice(s1, (0, 0), (H + 5, W)),
                     jax.lax.slice(s1, (2, 0), (H + 7, W)))
    s3 = jnp.maximum(jax.lax.slice(s2, (0, 0), (H + 1, W)),
                     jax.lax.slice(s2, (4, 0), (H + 5, W)))
    rowp = jnp.maximum(jax.lax.slice(s3, (0, 0), (H, W)),
                       jax.lax.slice(xp, (8, 0), (H + 8, W)))

    yp = jnp.concatenate(
        [jnp.full((H, pad), neg, jnp.float32), rowp, jnp.full((H, pad), neg, jnp.float32)],
        axis=1,
    )
    t1 = jnp.maximum(jax.lax.slice(yp, (0, 0), (H, W + 7)),
                     jax.lax.slice(yp, (0, 1), (H, W + 8)))
    t2 = jnp.maximum(jax.lax.slice(t1, (0, 0), (H, W + 5)),
                     jax.lax.slice(t1, (0, 2), (H, W + 7)))
    t3 = jnp.maximum(jax.lax.slice(t2, (0, 0), (H, W + 1)),
                     jax.lax.slice(t2, (0, 4), (H, W + 5)))
    pooled = jnp.maximum(jax.lax.slice(t3, (0, 0), (H, W)),
                         jax.lax.slice(yp, (0, 8), (H, W + 8)))

    peaks = (hm == pooled) & (hm > CONF_THRESHOLD)
    scores = jnp.where(peaks, hm, 0.0)
    sc_ref[...] = scores
    rowmax = jnp.max(scores, axis=1, keepdims=True)

    riota = jax.lax.broadcasted_iota(jnp.int32, (H, 1), 0)
    ciota = jax.lax.broadcasted_iota(jnp.int32, (1, W), 1)
    big = jnp.int32(2**30)
    inv_step = jnp.float32(1.0) / jnp.float32(W - 1)

    rix = jax.lax.broadcasted_iota(jnp.int32, (8, 128), 0)
    cix = jax.lax.broadcasted_iota(jnp.int32, (8, 128), 1)
    out = jnp.zeros((8, 128), jnp.float32)
    for k in range(MAX_FLARES):
        m = jnp.max(rowmax)
        r = jnp.min(jnp.where(rowmax == m, riota, big))
        row = sc_ref[pl.ds(r, 1), :]
        c = jnp.min(jnp.where(row == m, ciota, big))
        valid = m >= CONF_THRESHOLD
        x = jnp.where(valid, c.astype(jnp.float32) * inv_step, -1.0)
        y = jnp.where(valid, r.astype(jnp.float32) * inv_step, -1.0)
        conf = jnp.where(valid, m, 0.0)
        sel = cix == k
        out = jnp.where((rix == 0) & sel, conf, out)
        out = jnp.where((rix == 1) & sel, x, out)
        out = jnp.where((rix == 2) & sel, y, out)
        if k + 1 < MAX_FLARES:
            newrow = jnp.where(ciota == c, 0.0, row)
            sc_ref[pl.ds(r, 1), :] = newrow
            rowmax = jnp.where(riota == r, jnp.max(newrow), rowmax)
    return out


N_PER = 4


L = 16
SC_IMGS = 4
RPT = 64
SLAB = 32
NV = 512 // L
SC_W = 512
_BIGI = 1 << 30


HPW = SC_W + 2 * L


def _sc_body(hm, out, inbuf, hpad, scb, cnd, csh, mrg, obuf):
    c = lax.axis_index("c")
    s = lax.axis_index("s")
    img = 2 * c + s // 8
    srow = (s % 8) * RPT

    lane = lax.iota(jnp.int32, L)
    neg = jnp.full((L,), -jnp.inf, jnp.float32)
    zero = jnp.zeros((L,), jnp.float32)

    for j_ in range(SLAB):
        hpad[pl.ds(j_ * HPW, L)] = neg
        hpad[pl.ds(j_ * HPW + L + SC_W, L)] = neg

    rmx = [zero, zero, zero, zero]
    for slab in range(2):
        base = srow + slab * SLAB
        if slab == 0:
            edge = s % 8 == 0

            @pl.when(edge)
            def _():
                pltpu.sync_copy(hm.at[pl.ds(img * SC_W * SC_W, 40 * SC_W)],
                                inbuf.at[pl.ds(8 * SC_W, 40 * SC_W)])
                for j_ in range(8):
                    for v_ in range(NV):
                        inbuf[pl.ds(j_ * SC_W + v_ * L, L)] = neg

            @pl.when(jnp.logical_not(edge))
            def _():
                pltpu.sync_copy(hm.at[pl.ds(img * SC_W * SC_W + (base - 8) * SC_W, 48 * SC_W)], inbuf)
        else:
            edge = s % 8 == 7

            @pl.when(edge)
            def _():
                pltpu.sync_copy(hm.at[pl.ds(img * SC_W * SC_W + (base - 8) * SC_W, 40 * SC_W)],
                                inbuf.at[pl.ds(0, 40 * SC_W)])
                for j_ in range(40, 48):
                    for v_ in range(NV):
                        inbuf[pl.ds(j_ * SC_W + v_ * L, L)] = neg

            @pl.when(jnp.logical_not(edge))
            def _():
                pltpu.sync_copy(hm.at[pl.ds(img * SC_W * SC_W + (base - 8) * SC_W, 48 * SC_W)], inbuf)

        def pool_row(j, rmx_c):
            lr = slab * SLAB + j
            for v in range(NV):
                vp = inbuf[pl.ds((j + 4) * SC_W + v * L, L)]
                for dy in range(5, 13):
                    vp = jnp.maximum(vp, inbuf[pl.ds((j + dy) * SC_W + v * L, L)])
                hpad[pl.ds(j * HPW + L + v * L, L)] = vp
            rm = zero
            for v in range(NV):
                off = j * HPW + L + v * L
                p = hpad[pl.ds(off - 4, L)]
                for dd in range(-3, 5):
                    p = jnp.maximum(p, hpad[pl.ds(off + dd, L)])
                h = inbuf[pl.ds((j + 8) * SC_W + v * L, L)]
                sv = jnp.where((h == p) & (h > CONF_THRESHOLD), h, 0.0)
                scb[pl.ds(lr * SC_W + v * L, L)] = sv
                rm = jnp.maximum(rm, sv)
            m = jnp.max(rm)
            ins = lane == lr % L
            g = lr // L
            return tuple(
                jnp.where(ins & (g == gi), m, rmx_c[gi]) for gi in range(4)
            )

        rmx = lax.fori_loop(0, SLAB, pool_row, tuple(rmx))

    cv = zero
    ci = jnp.zeros((L,), jnp.int32)
    g0, g1, g2, g3 = rmx
    for k in range(MAX_FLARES):
        m = jnp.max(jnp.maximum(jnp.maximum(g0, g1), jnp.maximum(g2, g3)))
        rr = jnp.min(jnp.minimum(
            jnp.minimum(jnp.where(g0 == m, lane, 4096),
                        jnp.where(g1 == m, lane + L, 4096)),
            jnp.minimum(jnp.where(g2 == m, lane + 2 * L, 4096),
                        jnp.where(g3 == m, lane + 3 * L, 4096))))
        cmin = jnp.full((L,), 99999, jnp.int32)
        for v in range(NV):
            sv = scb[pl.ds(rr * SC_W + v * L, L)]
            cmin = jnp.minimum(cmin, jnp.where(sv == m, lane + v * L, 99999))
        cc = jnp.min(cmin)
        cv = jnp.where(lane == k, m, cv)
        ci = jnp.where(lane == k, (srow + rr) * SC_W + cc, ci)
        if k + 1 < MAX_FLARES:
            blk = (cc // L) * L
            sv = scb[pl.ds(rr * SC_W + blk, L)]
            scb[pl.ds(rr * SC_W + blk, L)] = jnp.where(lane + blk == cc, 0.0, sv)
            nrm = zero
            for v in range(NV):
                nrm = jnp.maximum(nrm, scb[pl.ds(rr * SC_W + v * L, L)])
            nm = jnp.max(nrm)
            ins = lane == rr % L
            g = rr // L
            g0 = jnp.where(ins & (g == 0), nm, g0)
            g1 = jnp.where(ins & (g == 1), nm, g1)
            g2 = jnp.where(ins & (g == 2), nm, g2)
            g3 = jnp.where(ins & (g == 3), nm, g3)

    cnd[pl.ds(0, L)] = cv
    cnd[pl.ds(L, L)] = plsc.bitcast(ci, jnp.float32)
    pltpu.sync_copy(cnd, csh.at[pl.ds(s * 2 * L, 2 * L)])
    plsc.subcore_barrier()

    @pl.when(s % 8 == 0)
    def _():
        pltpu.sync_copy(csh.at[pl.ds(s * 2 * L, 8 * 2 * L)], mrg)
        mv = [mrg[pl.ds(t * 2 * L, L)] for t in range(8)]
        mi = [plsc.bitcast(mrg[pl.ds(t * 2 * L + L, L)], jnp.int32) for t in range(8)]
        confv = zero
        xv = zero
        yv = zero
        inv_step = jnp.float32(1.0) / jnp.float32(SC_W - 1)
        for k in range(MAX_FLARES):
            acc = mv[0]
            for t in range(1, 8):
                acc = jnp.maximum(acc, mv[t])
            m = jnp.max(acc)
            iacc = jnp.where(mv[0] == m, mi[0], _BIGI)
            for t in range(1, 8):
                iacc = jnp.minimum(iacc, jnp.where(mv[t] == m, mi[t], _BIGI))
            li = jnp.min(iacc)
            valid = m >= CONF_THRESHOLD
            confv = jnp.where(lane == k, jnp.where(valid, m, 0.0), confv)
            xs = (li % SC_W).astype(jnp.float32) * inv_step
            ys = (li // SC_W).astype(jnp.float32) * inv_step
            xv = jnp.where(lane == k, jnp.where(valid, xs, -1.0), xv)
            yv = jnp.where(lane == k, jnp.where(valid, ys, -1.0), yv)
            if k + 1 < MAX_FLARES:
                mv = [jnp.where(mi[t] == li, -1.0, mv[t]) for t in range(8)]
        obuf[0, pl.ds(0, L)] = confv
        obuf[1, pl.ds(0, L)] = xv
        obuf[2, pl.ds(0, L)] = yv
        pltpu.sync_copy(obuf, out.at[img])


@functools.partial(
    pl.kernel,
    mesh=plsc.VectorSubcoreMesh(core_axis_name="c", subcore_axis_name="s"),
    compiler_params=pltpu.CompilerParams(needs_layout_passes=False),
    out_type=jax.ShapeDtypeStruct((SC_IMGS, 8, 128), jnp.float32),
    scratch_types=[
        pltpu.VMEM((48 * SC_W,), jnp.float32),
        pltpu.VMEM((SLAB * HPW,), jnp.float32),
        pltpu.VMEM((RPT * SC_W,), jnp.float32),
        pltpu.VMEM((2 * L,), jnp.float32),
        pltpu.VMEM_SHARED((L * 2 * L,), jnp.float32),
        pltpu.VMEM((8 * 2 * L,), jnp.float32),
        pltpu.VMEM((8, 128), jnp.float32),
    ],
)
def _sc_call(hm, out, inbuf, hpad, scb, cnd, csh, mrg, obuf):
    _sc_body(hm, out, inbuf, hpad, scb, cnd, csh, mrg, obuf)


def _poshead_kernel(hm_ref, out_ref, sc_ref):
    for j in range(N_PER):
        out_ref[j] = _poshead_one(hm_ref[j], sc_ref.at[j])


@jax.jit
def kernel(heatmap):
    B, _, H, W = heatmap.shape
    hm = heatmap.reshape(B, H, W)
    sc_out = _sc_call(hm[:SC_IMGS].reshape(-1))
    n_tc = B - SC_IMGS
    tc_out = pl.pallas_call(
        _poshead_kernel,
        grid=(n_tc // N_PER,),
        in_specs=[pl.BlockSpec((N_PER, H, W), lambda b: (b + SC_IMGS // N_PER, 0, 0))],
        out_specs=pl.BlockSpec((N_PER, 8, 128), lambda b: (b, 0, 0)),
        out_shape=jax.ShapeDtypeStruct((n_tc, 8, 128), jnp.float32),
        scratch_shapes=[pltpu.VMEM((N_PER, H, W), jnp.float32)],
    )(hm)
    out = jnp.concatenate([sc_out, tc_out], axis=0)
    conf = out[:, 0, :MAX_FLARES]
    pos = jnp.stack([out[:, 1, :MAX_FLARES], out[:, 2, :MAX_FLARES]], axis=-1)
    return pos, conf


# speedup vs baseline: 1.2406x; 1.0159x over previous
import functools

import jax
import jax.numpy as jnp
from jax import lax
from jax.experimental import pallas as pl
from jax.experimental.pallas import tpu as pltpu
from jax.experimental.pallas import tpu_sc as plsc

MAX_FLARES = 6
NMS_KERNEL = 9
CONF_THRESHOLD = 0.3


def _poshead_one(hm, sc_ref):
    H, W = hm.shape
    pad = NMS_KERNEL // 2
    neg = jnp.float32(-jnp.inf)

    xp = jnp.concatenate(
        [jnp.full((pad, W), neg, jnp.float32), hm, jnp.full((pad, W), neg, jnp.float32)],
        axis=0,
    )
    s1 = jnp.maximum(jax.lax.slice(xp, (0, 0), (H + 7, W)),
                     jax.lax.slice(xp, (1, 0), (H + 8, W)))
    s2 = jnp.maximum(jax.lax.slice(s1, (0, 0), (H + 5, W)),
                     jax.lax.slice(s1, (2, 0), (H + 7, W)))
    s3 = jnp.maximum(jax.lax.slice(s2, (0, 0), (H + 1, W)),
                     jax.lax.slice(s2, (4, 0), (H + 5, W)))
    rowp = jnp.maximum(jax.lax.slice(s3, (0, 0), (H, W)),
                       jax.lax.slice(xp, (8, 0), (H + 8, W)))

    yp = jnp.concatenate(
        [jnp.full((H, pad), neg, jnp.float32), rowp, jnp.full((H, pad), neg, jnp.float32)],
        axis=1,
    )
    t1 = jnp.maximum(jax.lax.slice(yp, (0, 0), (H, W + 7)),
                     jax.lax.slice(yp, (0, 1), (H, W + 8)))
    t2 = jnp.maximum(jax.lax.slice(t1, (0, 0), (H, W + 5)),
                     jax.lax.slice(t1, (0, 2), (H, W + 7)))
    t3 = jnp.maximum(jax.lax.slice(t2, (0, 0), (H, W + 1)),
                     jax.lax.slice(t2, (0, 4), (H, W + 5)))
    pooled = jnp.maximum(jax.lax.slice(t3, (0, 0), (H, W)),
                         jax.lax.slice(yp, (0, 8), (H, W + 8)))

    peaks = (hm == pooled) & (hm > CONF_THRESHOLD)
    scores = jnp.where(peaks, hm, 0.0)
    sc_ref[...] = scores
    rowmax = jnp.max(scores, axis=1, keepdims=True)

    riota = jax.lax.broadcasted_iota(jnp.int32, (H, 1), 0)
    ciota = jax.lax.broadcasted_iota(jnp.int32, (1, W), 1)
    big = jnp.int32(2**30)
    inv_step = jnp.float32(1.0) / jnp.float32(W - 1)

    rix = jax.lax.broadcasted_iota(jnp.int32, (8, 128), 0)
    cix = jax.lax.broadcasted_iota(jnp.int32, (8, 128), 1)
    out = jnp.zeros((8, 128), jnp.float32)
    for k in range(MAX_FLARES):
        m = jnp.max(rowmax)
        r = jnp.min(jnp.where(rowmax == m, riota, big))
        row = sc_ref[pl.ds(r, 1), :]
        c = jnp.min(jnp.where(row == m, ciota, big))
        valid = m >= CONF_THRESHOLD
        x = jnp.where(valid, c.astype(jnp.float32) * inv_step, -1.0)
        y = jnp.where(valid, r.astype(jnp.float32) * inv_step, -1.0)
        conf = jnp.where(valid, m, 0.0)
        sel = cix == k
        out = jnp.where((rix == 0) & sel, conf, out)
        out = jnp.where((rix == 1) & sel, x, out)
        out = jnp.where((rix == 2) & sel, y, out)
        if k + 1 < MAX_FLARES:
            newrow = jnp.where(ciota == c, 0.0, row)
            sc_ref[pl.ds(r, 1), :] = newrow
            rowmax = jnp.where(riota == r, jnp.max(newrow), rowmax)
    return out


N_PER = 4


L = 16
SC_IMGS = 4
RPT = 64
SLAB = 32
NV = 512 // L
SC_W = 512
_BIGI = 1 << 30


HPW = SC_W + 2 * L


def _sc_body(hm, out, inbuf, hpad, scb, cnd, csh, mrg, obuf):
    c = lax.axis_index("c")
    s = lax.axis_index("s")
    img = 2 * c + s // 8
    srow = (s % 8) * RPT

    lane = lax.iota(jnp.int32, L)
    neg = jnp.full((L,), -jnp.inf, jnp.float32)
    zero = jnp.zeros((L,), jnp.float32)

    for j_ in range(SLAB):
        hpad[j_, pl.ds(0, L)] = neg
        hpad[j_, pl.ds(L + SC_W, L)] = neg

    rmx = [zero, zero, zero, zero]
    for slab in range(2):
        base = srow + slab * SLAB
        if slab == 0:
            edge = s % 8 == 0

            @pl.when(edge)
            def _():
                pltpu.sync_copy(hm.at[img, pl.ds(0, 40), :], inbuf.at[pl.ds(8, 40)])
                for j_ in range(8):
                    for v_ in range(NV):
                        inbuf[j_, pl.ds(v_ * L, L)] = neg

            @pl.when(jnp.logical_not(edge))
            def _():
                pltpu.sync_copy(hm.at[img, pl.ds(base - 8, 48), :], inbuf)
        else:
            edge = s % 8 == 7

            @pl.when(edge)
            def _():
                pltpu.sync_copy(hm.at[img, pl.ds(base - 8, 40), :], inbuf.at[pl.ds(0, 40)])
                for j_ in range(40, 48):
                    for v_ in range(NV):
                        inbuf[j_, pl.ds(v_ * L, L)] = neg

            @pl.when(jnp.logical_not(edge))
            def _():
                pltpu.sync_copy(hm.at[img, pl.ds(base - 8, 48), :], inbuf)

        def pool_row(j, rmx_c):
            lr = slab * SLAB + j
            for v in range(NV):
                vp = inbuf[j + 4, pl.ds(v * L, L)]
                for dy in range(5, 13):
                    vp = jnp.maximum(vp, inbuf[j + dy, pl.ds(v * L, L)])
                hpad[j, pl.ds(L + v * L, L)] = vp
            rm = zero
            for v in range(NV):
                off = L + v * L
                p = hpad[j, pl.ds(off - 4, L)]
                for dd in range(-3, 5):
                    p = jnp.maximum(p, hpad[j, pl.ds(off + dd, L)])
                h = inbuf[j + 8, pl.ds(v * L, L)]
                sv = jnp.where((h == p) & (h > CONF_THRESHOLD), h, 0.0)
                scb[lr, pl.ds(v * L, L)] = sv
                rm = jnp.maximum(rm, sv)
            m = jnp.max(rm)
            ins = lane == lr % L
            g = lr // L
            return tuple(
                jnp.where(ins & (g == gi), m, rmx_c[gi]) for gi in range(4)
            )

        rmx = lax.fori_loop(0, SLAB, pool_row, tuple(rmx))

    cv = zero
    ci = jnp.zeros((L,), jnp.int32)
    g0, g1, g2, g3 = rmx
    for k in range(MAX_FLARES):
        m = jnp.max(jnp.maximum(jnp.maximum(g0, g1), jnp.maximum(g2, g3)))
        rr = jnp.min(jnp.minimum(
            jnp.minimum(jnp.where(g0 == m, lane, 4096),
                        jnp.where(g1 == m, lane + L, 4096)),
            jnp.minimum(jnp.where(g2 == m, lane + 2 * L, 4096),
                        jnp.where(g3 == m, lane + 3 * L, 4096))))
        cmin = jnp.full((L,), 99999, jnp.int32)
        for v in range(NV):
            sv = scb[rr, pl.ds(v * L, L)]
            cmin = jnp.minimum(cmin, jnp.where(sv == m, lane + v * L, 99999))
        cc = jnp.min(cmin)
        cv = jnp.where(lane == k, m, cv)
        ci = jnp.where(lane == k, (srow + rr) * SC_W + cc, ci)
        if k + 1 < MAX_FLARES:
            blk = (cc // L) * L
            sv = scb[rr, pl.ds(blk, L)]
            scb[rr, pl.ds(blk, L)] = jnp.where(lane + blk == cc, 0.0, sv)
            nrm = zero
            for v in range(NV):
                nrm = jnp.maximum(nrm, scb[rr, pl.ds(v * L, L)])
            nm = jnp.max(nrm)
            ins = lane == rr % L
            g = rr // L
            g0 = jnp.where(ins & (g == 0), nm, g0)
            g1 = jnp.where(ins & (g == 1), nm, g1)
            g2 = jnp.where(ins & (g == 2), nm, g2)
            g3 = jnp.where(ins & (g == 3), nm, g3)

    cnd[pl.ds(0, L)] = cv
    cnd[pl.ds(L, L)] = plsc.bitcast(ci, jnp.float32)
    pltpu.sync_copy(cnd, csh.at[pl.ds(s * 2 * L, 2 * L)])
    plsc.subcore_barrier()

    @pl.when(s % 8 == 0)
    def _():
        pltpu.sync_copy(csh.at[pl.ds(s * 2 * L, 8 * 2 * L)], mrg)
        mv = [mrg[pl.ds(t * 2 * L, L)] for t in range(8)]
        mi = [plsc.bitcast(mrg[pl.ds(t * 2 * L + L, L)], jnp.int32) for t in range(8)]
        confv = zero
        xv = zero
        yv = zero
        inv_step = jnp.float32(1.0) / jnp.float32(SC_W - 1)
        for k in range(MAX_FLARES):
            acc = mv[0]
            for t in range(1, 8):
                acc = jnp.maximum(acc, mv[t])
            m = jnp.max(acc)
            iacc = jnp.where(mv[0] == m, mi[0], _BIGI)
            for t in range(1, 8):
                iacc = jnp.minimum(iacc, jnp.where(mv[t] == m, mi[t], _BIGI))
            li = jnp.min(iacc)
            valid = m >= CONF_THRESHOLD
            confv = jnp.where(lane == k, jnp.where(valid, m, 0.0), confv)
            xs = (li % SC_W).astype(jnp.float32) * inv_step
            ys = (li // SC_W).astype(jnp.float32) * inv_step
            xv = jnp.where(lane == k, jnp.where(valid, xs, -1.0), xv)
            yv = jnp.where(lane == k, jnp.where(valid, ys, -1.0), yv)
            if k + 1 < MAX_FLARES:
                mv = [jnp.where(mi[t] == li, -1.0, mv[t]) for t in range(8)]
        obuf[0, pl.ds(0, L)] = confv
        obuf[1, pl.ds(0, L)] = xv
        obuf[2, pl.ds(0, L)] = yv
        pltpu.sync_copy(obuf, out.at[img])


@functools.partial(
    pl.kernel,
    mesh=plsc.VectorSubcoreMesh(core_axis_name="c", subcore_axis_name="s"),
    compiler_params=pltpu.CompilerParams(needs_layout_passes=False),
    out_type=jax.ShapeDtypeStruct((SC_IMGS, 8, 128), jnp.float32),
    scratch_types=[
        pltpu.VMEM((48, SC_W), jnp.float32),
        pltpu.VMEM((SLAB, HPW), jnp.float32),
        pltpu.VMEM((RPT, SC_W), jnp.float32),
        pltpu.VMEM((2 * L,), jnp.float32),
        pltpu.VMEM_SHARED((L * 2 * L,), jnp.float32),
        pltpu.VMEM((8 * 2 * L,), jnp.float32),
        pltpu.VMEM((8, 128), jnp.float32),
    ],
)
def _sc_call(hm, out, inbuf, hpad, scb, cnd, csh, mrg, obuf):
    _sc_body(hm, out, inbuf, hpad, scb, cnd, csh, mrg, obuf)


def _poshead_kernel(hm_ref, out_ref, sc_ref):
    for j in range(N_PER):
        out_ref[j] = _poshead_one(hm_ref[j], sc_ref.at[j])


@jax.jit
def kernel(heatmap):
    B, _, H, W = heatmap.shape
    hm = heatmap.reshape(B, H, W)
    sc_out = _sc_call(hm)
    n_tc = B - SC_IMGS
    tc_out = pl.pallas_call(
        _poshead_kernel,
        grid=(n_tc // N_PER,),
        in_specs=[pl.BlockSpec((N_PER, H, W), lambda b: (b + SC_IMGS // N_PER, 0, 0))],
        out_specs=pl.BlockSpec((N_PER, 8, 128), lambda b: (b, 0, 0)),
        out_shape=jax.ShapeDtypeStruct((n_tc, 8, 128), jnp.float32),
        scratch_shapes=[pltpu.VMEM((N_PER, H, W), jnp.float32)],
    )(hm)
    out = jnp.concatenate([sc_out, tc_out], axis=0)
    conf = out[:, 0, :MAX_FLARES]
    pos = jnp.stack([out[:, 1, :MAX_FLARES], out[:, 2, :MAX_FLARES]], axis=-1)
    return pos, conf
